# baseline (speedup 1.0000x reference)
import functools

import jax
import jax.numpy as jnp
from jax import lax
from jax.experimental import pallas as pl
from jax.experimental.pallas import tpu as pltpu
from jax.experimental.pallas import tpu_sc as plsc

_NC = 2
_NS = 16
_NW = _NC * _NS
_L = 16


def _sc_body(gd, ng, d4, row_hbm, col_hbm, io_hbm, pair_hbm, out_hbm,
             row_v, col_v, io_v, pair_v, buf0, buf1, sem0, sem1):
    half_rows = gd // 2
    slabs_per_w = (ng * gd) // _NW
    n_half = 2 * slabs_per_w
    wid = lax.axis_index("s") * _NC + lax.axis_index("c")

    pltpu.sync_copy(row_hbm, row_v)
    pltpu.sync_copy(col_hbm, col_v)
    pltpu.sync_copy(io_hbm, io_v)
    pltpu.sync_copy(pair_hbm, pair_v)

    def build_and_fire(j, buf, sem):
        slab = wid * slabs_per_w + lax.div(j, 2)
        half = lax.rem(j, 2)
        g = lax.div(slab, gd)
        r = lax.rem(slab, gd)
        io_i = lax.rem(g, 2)
        pair_i = lax.div(g, 2)
        c0 = half * half_rows

        def row_body(cl, carry):
            for k in range(d4 // _L):
                buf[cl, pl.ds(k * _L, _L)] = row_v[r, pl.ds(k * _L, _L)]
            for k in range(d4 // _L):
                buf[cl, pl.ds(d4 + k * _L, _L)] = (
                    col_v[c0 + cl, pl.ds(k * _L, _L)])
            for k in range(d4 // _L):
                buf[cl, pl.ds(2 * d4 + k * _L, _L)] = (
                    io_v[io_i, pl.ds(k * _L, _L)])
            for k in range(d4 // _L):
                buf[cl, pl.ds(3 * d4 + k * _L, _L)] = (
                    pair_v[pair_i, pl.ds(k * _L, _L)])
            return carry

        lax.fori_loop(0, half_rows, row_body, 0)
        return pltpu.async_copy(buf, out_hbm.at[g, r, pl.ds(c0, half_rows)],
                                sem)

    build_and_fire(0, buf0, sem0)
    build_and_fire(1, buf1, sem1)

    def pair_body(jj, carry):
        pltpu.make_async_copy(
            buf0, out_hbm.at[0, 0, pl.ds(0, half_rows)], sem0).wait()
        build_and_fire(2 * jj, buf0, sem0)
        pltpu.make_async_copy(
            buf1, out_hbm.at[0, 0, pl.ds(0, half_rows)], sem1).wait()
        build_and_fire(2 * jj + 1, buf1, sem1)
        return carry

    lax.fori_loop(1, n_half // 2, pair_body, 0)
    pltpu.make_async_copy(
        buf0, out_hbm.at[0, 0, pl.ds(0, half_rows)], sem0).wait()
    pltpu.make_async_copy(
        buf1, out_hbm.at[0, 0, pl.ds(0, half_rows)], sem1).wait()


def kernel(row_table, col_table, io_table, pair_table, num_grids, grid_dim):
    gd = row_table.shape[0]
    ng = pair_table.shape[0] - 1
    d4 = row_table.shape[-1]
    d = 4 * d4
    half_rows = gd // 2

    mesh = plsc.VectorSubcoreMesh(core_axis_name="c", subcore_axis_name="s")
    sc_fn = pl.kernel(
        functools.partial(_sc_body, gd, ng, d4),
        mesh=mesh,
        out_type=jax.ShapeDtypeStruct((ng, gd, gd, d), row_table.dtype),
        scratch_types=[
            pltpu.VMEM((gd, d4), jnp.float32),
            pltpu.VMEM((gd, d4), jnp.float32),
            pltpu.VMEM(io_table.shape, jnp.float32),
            pltpu.VMEM(pair_table.shape, jnp.float32),
            pltpu.VMEM((half_rows, d), jnp.float32),
            pltpu.VMEM((half_rows, d), jnp.float32),
            pltpu.SemaphoreType.DMA,
            pltpu.SemaphoreType.DMA,
        ],
    )
    return sc_fn(row_table, col_table, io_table, pair_table)


# speedup vs baseline: 4.1976x; 4.1976x over previous
import functools

import jax
import jax.numpy as jnp
from jax import lax
from jax.experimental import pallas as pl
from jax.experimental.pallas import tpu as pltpu
from jax.experimental.pallas import tpu_sc as plsc

_NC = 2
_NS = 16
_NW = _NC * _NS
_L = 16


def _replicate(src_ref, src_row, dst_ref, gd, d4):
    vecs = [src_ref[src_row, pl.ds(k * _L, _L)] for k in range(d4 // _L)]

    def body(c, carry):
        for k in range(d4 // _L):
            dst_ref[c, pl.ds(k * _L, _L)] = vecs[k]
        return carry

    lax.fori_loop(0, gd, body, 0)


def _sc_body(gd, ng, d4, row_hbm, col_hbm, io_hbm, pair_hbm, out_hbm,
             col_v, io_s, pair_s, row_rep0, row_rep1, io_rep0, io_rep1,
             pair_rep0, pair_rep1, sem):
    r_per_w = gd // _NW
    wid = lax.axis_index("s") * _NC + lax.axis_index("c")
    r0 = wid * r_per_w

    pltpu.sync_copy(col_hbm, col_v)
    pltpu.sync_copy(io_hbm, io_s)
    pltpu.sync_copy(pair_hbm, pair_s)

    row_reps = [row_rep0, row_rep1]
    for rl in range(r_per_w):
        pltpu.sync_copy(row_hbm.at[r0 + rl], row_reps[rl].at[0])
        _replicate(row_reps[rl], 0, row_reps[rl], gd, d4)
    _replicate(io_s, 0, io_rep0, gd, d4)
    _replicate(io_s, 1, io_rep1, gd, d4)
    io_reps = [io_rep0, io_rep1]
    pair_reps = [pair_rep0, pair_rep1]

    def drain_one():
        pltpu.make_async_copy(
            col_v, out_hbm.at[0, 0, :, pl.ds(0, d4)], sem).wait()

    gen_fired = [0, 0]
    for g in range(ng):
        if g % 2 == 0:
            pb = (g // 2) % 2
            for _ in range(gen_fired[pb]):
                drain_one()
            gen_fired[pb] = 0
            _replicate(pair_s, g // 2, pair_reps[pb], gd, d4)
        pb = (g // 2) % 2
        for rl in range(r_per_w):
            r = r0 + rl
            pltpu.async_copy(
                row_reps[rl], out_hbm.at[g, r, :, pl.ds(0, d4)], sem)
            pltpu.async_copy(
                col_v, out_hbm.at[g, r, :, pl.ds(d4, d4)], sem)
            pltpu.async_copy(
                io_reps[g % 2], out_hbm.at[g, r, :, pl.ds(2 * d4, d4)], sem)
            pltpu.async_copy(
                pair_reps[pb], out_hbm.at[g, r, :, pl.ds(3 * d4, d4)], sem)
            gen_fired[pb] += 4
    for _ in range(gen_fired[0] + gen_fired[1]):
        drain_one()


def kernel(row_table, col_table, io_table, pair_table, num_grids, grid_dim):
    gd = row_table.shape[0]
    ng = pair_table.shape[0] - 1
    d4 = row_table.shape[-1]
    d = 4 * d4

    mesh = plsc.VectorSubcoreMesh(core_axis_name="c", subcore_axis_name="s")
    tile = pltpu.VMEM((gd, d4), jnp.float32)
    sc_fn = pl.kernel(
        functools.partial(_sc_body, gd, ng, d4),
        mesh=mesh,
        out_type=jax.ShapeDtypeStruct((ng, gd, gd, d), row_table.dtype),
        scratch_types=[
            tile,
            pltpu.VMEM(io_table.shape, jnp.float32),
            pltpu.VMEM(pair_table.shape, jnp.float32),
            tile, tile,
            tile, tile,
            tile, tile,
            pltpu.SemaphoreType.DMA,
        ],
    )
    return sc_fn(row_table, col_table, io_table, pair_table)
